# baseline (speedup 1.0000x reference)
import dataclasses

import jax
import jax.numpy as jnp
from jax import lax
from jax.experimental import pallas as pl
from jax.experimental.pallas import tpu as pltpu
from jax.experimental.pallas import tpu_sc as plsc

B = 16384
V = 1000000
D = 32
L = 50
NC = 2
NS = 16
NW = NC * NS
BPW = B // NW
C = 64
NCHUNK = BPW // C
RPC = C * L
GW = 128
NGW = RPC // GW

VB = 512
NBF = V // VB
TAIL = V - NBF * VB
TV0 = NBF * VB
KMAX = -(-NBF // NW)


def _tr_body(wtu_hbm, wti_hbm, wth_hbm, fu_hbm, fi_hbm, fh_hbm,
             inb0, inb1, inb2, outb0, outb1, outb2, intl,
             si0, si1, si2, so0, so1, so2):
    wid = lax.axis_index("s") * NC + lax.axis_index("c")
    inbs, outbs = (inb0, inb1, inb2), (outb0, outb1, outb2)
    sis, sos = (si0, si1, si2), (so0, so1, so2)
    dlanes = lax.iota(jnp.int32, 16) * D

    def _transpose_block(src, dst, n):
        @pl.loop(0, n, step=16)
        def _v(vv):
            ibase = dlanes + vv * D
            for d in range(D):
                plsc.store_scatter(dst, [ibase + d], src[d, pl.ds(vv, 16)])

    def _fire_in(win, i, p):
        for g in range(D // 8):
            pltpu.async_copy(win.at[pl.ds(8 * g, 8), pl.ds(i * VB, VB)],
                             inbs[p].at[pl.ds(8 * g, 8), :], sis[p])

    def _wait_in(win, p):
        for g in range(D // 8):
            pltpu.make_async_copy(win.at[pl.ds(8 * g, 8), pl.ds(0, VB)],
                                  inbs[p].at[pl.ds(8 * g, 8), :],
                                  sis[p]).wait()

    def _wait_out(wout, p):
        pltpu.make_async_copy(outbs[p], wout.at[pl.ds(0, VB * D)],
                              sos[p]).wait()

    RING = 3
    for win, wout in ((wtu_hbm, fu_hbm), (wti_hbm, fi_hbm), (wth_hbm, fh_hbm)):
        for p in range(RING):
            @pl.when(wid + p * NW < NBF)
            def _prime():
                _fire_in(win, wid + p * NW, p)

        @pl.loop(0, RING * KMAX, step=RING)
        def _blk(k):
            for p in range(RING):
                idx = wid + (k + p) * NW

                @pl.when(idx < NBF)
                def _one():
                    _wait_in(win, p)

                    @pl.when(k + p >= RING)
                    def _drain():
                        _wait_out(wout, p)

                    _transpose_block(inbs[p], outbs[p], VB)
                    pltpu.async_copy(
                        outbs[p], wout.at[pl.ds(idx * VB * D, VB * D)],
                        sos[p])

                    @pl.when(idx + RING * NW < NBF)
                    def _more():
                        _fire_in(win, idx + RING * NW, p)

        for p in range(RING):
            @pl.when(wid + p * NW < NBF)
            def _drain_last():
                _wait_out(wout, p)

        @pl.when(wid == NW - 1)
        def _tail():
            pltpu.sync_copy(win.at[:, pl.ds(TV0, TAIL)], intl)
            _transpose_block(intl, outb0, TAIL)
            pltpu.sync_copy(outb0.at[pl.ds(0, TAIL * D)],
                            wout.at[pl.ds(TV0 * D, TAIL * D)])


def _sc_body(uid_hbm, iid_hbm, price_hbm, hidx_hbm, wu_hbm, wi_hbm, wh_hbm,
             wp_hbm, out_hbm,
             uidx, iidx, pbuf, hidx, ubuf, ibuf, hbuf, cntb, outb, wpb, sem):
    wid = lax.axis_index("s") * NC + lax.axis_index("c")
    base = wid * BPW
    pltpu.sync_copy(wp_hbm, wpb)
    wp0 = wpb[pl.ds(0, 16)]
    wp1 = wpb[pl.ds(16, 16)]
    lanes = lax.iota(jnp.int32, 16)

    @pl.loop(0, NCHUNK)
    def _chunk(c):
        cb = base + c * C
        pltpu.sync_copy(uid_hbm.at[pl.ds(cb, C)], uidx)
        pltpu.sync_copy(iid_hbm.at[pl.ds(cb, C)], iidx)
        pltpu.sync_copy(price_hbm.at[pl.ds(cb, C)], pbuf)
        pltpu.sync_copy(hidx_hbm.at[pl.ds(cb * L, RPC)], hidx)

        cps = [pltpu.async_copy(wu_hbm.at[uidx], ubuf, sem),
               pltpu.async_copy(wi_hbm.at[iidx], ibuf, sem)]
        for j in range(NGW):
            cps.append(pltpu.async_copy(
                wh_hbm.at[hidx.at[pl.ds(j * GW, GW)]],
                hbuf.at[pl.ds(j * GW, GW)], sem))

        for g in range(C // 16):
            cnt = jnp.zeros((16,), jnp.float32)
            lbase = lanes * L + (g * 16 * L)
            for k in range(L):
                iv = plsc.load_gather(hidx, [lbase + k])
                cnt = cnt + jnp.where(iv != 0, 1.0, 0.0)
            cntb[pl.ds(g * 16, 16)] = jnp.maximum(cnt, 1.0)

        for cp in cps:
            cp.wait()

        @pl.loop(0, C)
        def _b(b):
            r0 = b * L
            a0 = hbuf[r0, pl.ds(0, 16)]
            a1 = hbuf[r0, pl.ds(16, 16)]
            for r in range(1, L):
                a0 = a0 + hbuf[r0 + r, pl.ds(0, 16)]
                a1 = a1 + hbuf[r0 + r, pl.ds(16, 16)]
            bidx = jnp.full((16,), b, jnp.int32)
            csp = plsc.load_gather(cntb, [bidx])
            psp = plsc.load_gather(pbuf, [bidx])
            outb[b, pl.ds(0, 16)] = ubuf[b, pl.ds(0, 16)]
            outb[b, pl.ds(16, 16)] = ubuf[b, pl.ds(16, 16)]
            outb[b, pl.ds(32, 16)] = ibuf[b, pl.ds(0, 16)]
            outb[b, pl.ds(48, 16)] = ibuf[b, pl.ds(16, 16)]
            outb[b, pl.ds(64, 16)] = psp * wp0
            outb[b, pl.ds(80, 16)] = psp * wp1
            outb[b, pl.ds(96, 16)] = a0 / csp
            outb[b, pl.ds(112, 16)] = a1 / csp

        pltpu.sync_copy(outb, out_hbm.at[pl.ds(cb, C)])


@jax.jit
def kernel(user_id, item_id, price, click_history, W_user, W_item, W_hist,
           W_price):
    uid = user_id.astype(jnp.int32)
    iid = item_id.astype(jnp.int32)
    hidx = click_history.astype(jnp.int32).reshape(-1)
    wp = W_price.astype(jnp.float32).reshape(-1)
    pricef = price.astype(jnp.float32)

    cp = pltpu.CompilerParams()
    if "needs_layout_passes" in pltpu.CompilerParams.__dataclass_fields__:
        cp = dataclasses.replace(cp, needs_layout_passes=False)
    cpu = dataclasses.replace(cp, use_tc_tiling_on_sc=False)
    mesh = plsc.VectorSubcoreMesh(core_axis_name="c", subcore_axis_name="s")

    fshape = jax.ShapeDtypeStruct((V * D,), jnp.float32)
    tr_run = pl.kernel(
        _tr_body,
        out_type=(fshape, fshape, fshape),
        mesh=mesh,
        compiler_params=cp,
        scratch_types=[
            pltpu.VMEM((D, VB), jnp.float32),
            pltpu.VMEM((D, VB), jnp.float32),
            pltpu.VMEM((D, VB), jnp.float32),
            pltpu.VMEM((VB * D,), jnp.float32),
            pltpu.VMEM((VB * D,), jnp.float32),
            pltpu.VMEM((VB * D,), jnp.float32),
            pltpu.VMEM((D, TAIL), jnp.float32),
            pltpu.SemaphoreType.DMA,
            pltpu.SemaphoreType.DMA,
            pltpu.SemaphoreType.DMA,
            pltpu.SemaphoreType.DMA,
            pltpu.SemaphoreType.DMA,
            pltpu.SemaphoreType.DMA,
        ],
    )
    fu, fi, fh = tr_run(W_user.T, W_item.T, W_hist.T)
    Wu = fu.reshape(V, D)
    Wi = fi.reshape(V, D)
    Wh = fh.reshape(V, D)

    run = pl.kernel(
        _sc_body,
        out_type=jax.ShapeDtypeStruct((B, 4 * D), jnp.float32),
        mesh=mesh,
        compiler_params=cpu,
        scratch_types=[
            pltpu.VMEM((C,), jnp.int32),
            pltpu.VMEM((C,), jnp.int32),
            pltpu.VMEM((C,), jnp.float32),
            pltpu.VMEM((RPC,), jnp.int32),
            pltpu.VMEM((C, D), jnp.float32),
            pltpu.VMEM((C, D), jnp.float32),
            pltpu.VMEM((RPC, D), jnp.float32),
            pltpu.VMEM((C,), jnp.float32),
            pltpu.VMEM((C, 4 * D), jnp.float32),
            pltpu.VMEM((D,), jnp.float32),
            pltpu.SemaphoreType.DMA,
        ],
    )
    return run(uid, iid, pricef, hidx, Wu, Wi, Wh, wp)


# speedup vs baseline: 2.2473x; 2.2473x over previous
import dataclasses

import jax
import jax.numpy as jnp
from jax import lax
from jax.experimental import pallas as pl
from jax.experimental.pallas import tpu as pltpu
from jax.experimental.pallas import tpu_sc as plsc

B = 16384
V = 1000000
D = 32
L = 50
NC = 2
NS = 16
NW = NC * NS
BPW = B // NW
C = 64
NCHUNK = BPW // C
RPC = C * L
GW = 128
NGW = RPC // GW

VB = 512
NBF = V // VB
TAIL = V - NBF * VB
TV0 = NBF * VB
KMAX = -(-NBF // NW)


def _tr_body(wtu_hbm, wti_hbm, wth_hbm, fu_hbm, fi_hbm, fh_hbm,
             inb0, inb1, inb2, outb0, outb1, outb2, intl,
             si0, si1, si2, so0, so1, so2):
    wid = lax.axis_index("s") * NC + lax.axis_index("c")
    inbs, outbs = (inb0, inb1, inb2), (outb0, outb1, outb2)
    sis, sos = (si0, si1, si2), (so0, so1, so2)
    lanes = lax.iota(jnp.int32, 16)

    def _transpose_block(src, dst, n):
        @pl.loop(0, n, step=16)
        def _v(vv):
            vvec = lanes + vv
            sbase = vvec * D
            for j in range(D):
                dvec = (lanes + j) % D
                g = plsc.load_gather(src, [dvec, vvec])
                plsc.store_scatter(dst, [sbase + dvec], g)

    def _fire_in(win, i, p):
        for g in range(D // 8):
            pltpu.async_copy(win.at[pl.ds(8 * g, 8), pl.ds(i * VB, VB)],
                             inbs[p].at[pl.ds(8 * g, 8), :], sis[p])

    def _wait_in(win, p):
        for g in range(D // 8):
            pltpu.make_async_copy(win.at[pl.ds(8 * g, 8), pl.ds(0, VB)],
                                  inbs[p].at[pl.ds(8 * g, 8), :],
                                  sis[p]).wait()

    def _wait_out(wout, p):
        pltpu.make_async_copy(outbs[p].at[pl.ds(0, VB * D)],
                              wout.at[pl.ds(0, VB * D)],
                              sos[p]).wait()

    RING = 3
    for win, wout in ((wtu_hbm, fu_hbm), (wti_hbm, fi_hbm), (wth_hbm, fh_hbm)):
        for p in range(RING):
            @pl.when(wid + p * NW < NBF)
            def _prime():
                _fire_in(win, wid + p * NW, p)

        @pl.loop(0, RING * KMAX, step=RING)
        def _blk(k):
            for p in range(RING):
                idx = wid + (k + p) * NW

                @pl.when(idx < NBF)
                def _one():
                    _wait_in(win, p)

                    @pl.when(k + p >= RING)
                    def _drain():
                        _wait_out(wout, p)

                    _transpose_block(inbs[p], outbs[p], VB)
                    pltpu.async_copy(
                        outbs[p].at[pl.ds(0, VB * D)],
                        wout.at[pl.ds(idx * VB * D, VB * D)],
                        sos[p])

                    @pl.when(idx + RING * NW < NBF)
                    def _more():
                        _fire_in(win, idx + RING * NW, p)

        for p in range(RING):
            @pl.when(wid + p * NW < NBF)
            def _drain_last():
                _wait_out(wout, p)

        @pl.when(wid == NW - 1)
        def _tail():
            pltpu.sync_copy(win.at[:, pl.ds(TV0, TAIL)], intl)
            _transpose_block(intl, outb0, TAIL)
            pltpu.sync_copy(outb0.at[pl.ds(0, TAIL * D)],
                            wout.at[pl.ds(TV0 * D, TAIL * D)])


def _sc_body(uid_hbm, iid_hbm, price_hbm, hidx_hbm, wu_hbm, wi_hbm, wh_hbm,
             wp_hbm, out_hbm,
             uidx, iidx, pbuf, hidx, ubuf, ibuf, hbuf, cntb, outb, wpb, sem):
    wid = lax.axis_index("s") * NC + lax.axis_index("c")
    base = wid * BPW
    pltpu.sync_copy(wp_hbm, wpb)
    wp0 = wpb[pl.ds(0, 16)]
    wp1 = wpb[pl.ds(16, 16)]
    lanes = lax.iota(jnp.int32, 16)

    @pl.loop(0, NCHUNK)
    def _chunk(c):
        cb = base + c * C
        pltpu.sync_copy(uid_hbm.at[pl.ds(cb, C)], uidx)
        pltpu.sync_copy(iid_hbm.at[pl.ds(cb, C)], iidx)
        pltpu.sync_copy(price_hbm.at[pl.ds(cb, C)], pbuf)
        pltpu.sync_copy(hidx_hbm.at[pl.ds(cb * L, RPC)], hidx)

        cps = [pltpu.async_copy(wu_hbm.at[uidx], ubuf, sem),
               pltpu.async_copy(wi_hbm.at[iidx], ibuf, sem)]
        for j in range(NGW):
            cps.append(pltpu.async_copy(
                wh_hbm.at[hidx.at[pl.ds(j * GW, GW)]],
                hbuf.at[pl.ds(j * GW, GW)], sem))

        for g in range(C // 16):
            cnt = jnp.zeros((16,), jnp.float32)
            lbase = lanes * L + (g * 16 * L)
            for k in range(L):
                iv = plsc.load_gather(hidx, [lbase + k])
                cnt = cnt + jnp.where(iv != 0, 1.0, 0.0)
            cntb[pl.ds(g * 16, 16)] = jnp.maximum(cnt, 1.0)

        for cp in cps:
            cp.wait()

        @pl.loop(0, C)
        def _b(b):
            r0 = b * L
            a0 = hbuf[r0, pl.ds(0, 16)]
            a1 = hbuf[r0, pl.ds(16, 16)]
            for r in range(1, L):
                a0 = a0 + hbuf[r0 + r, pl.ds(0, 16)]
                a1 = a1 + hbuf[r0 + r, pl.ds(16, 16)]
            bidx = jnp.full((16,), b, jnp.int32)
            csp = plsc.load_gather(cntb, [bidx])
            psp = plsc.load_gather(pbuf, [bidx])
            outb[b, pl.ds(0, 16)] = ubuf[b, pl.ds(0, 16)]
            outb[b, pl.ds(16, 16)] = ubuf[b, pl.ds(16, 16)]
            outb[b, pl.ds(32, 16)] = ibuf[b, pl.ds(0, 16)]
            outb[b, pl.ds(48, 16)] = ibuf[b, pl.ds(16, 16)]
            outb[b, pl.ds(64, 16)] = psp * wp0
            outb[b, pl.ds(80, 16)] = psp * wp1
            outb[b, pl.ds(96, 16)] = a0 / csp
            outb[b, pl.ds(112, 16)] = a1 / csp

        pltpu.sync_copy(outb, out_hbm.at[pl.ds(cb, C)])


@jax.jit
def kernel(user_id, item_id, price, click_history, W_user, W_item, W_hist,
           W_price):
    uid = user_id.astype(jnp.int32)
    iid = item_id.astype(jnp.int32)
    hidx = click_history.astype(jnp.int32).reshape(-1)
    wp = W_price.astype(jnp.float32).reshape(-1)
    pricef = price.astype(jnp.float32)

    cp = pltpu.CompilerParams()
    if "needs_layout_passes" in pltpu.CompilerParams.__dataclass_fields__:
        cp = dataclasses.replace(cp, needs_layout_passes=False)
    cpu = dataclasses.replace(cp, use_tc_tiling_on_sc=False)
    mesh = plsc.VectorSubcoreMesh(core_axis_name="c", subcore_axis_name="s")

    fshape = jax.ShapeDtypeStruct((V * D,), jnp.float32)
    tr_run = pl.kernel(
        _tr_body,
        out_type=(fshape, fshape, fshape),
        mesh=mesh,
        compiler_params=cp,
        scratch_types=[
            pltpu.VMEM((D, VB), jnp.float32),
            pltpu.VMEM((D, VB), jnp.float32),
            pltpu.VMEM((D, VB), jnp.float32),
            pltpu.VMEM((VB * D,), jnp.float32),
            pltpu.VMEM((VB * D,), jnp.float32),
            pltpu.VMEM((VB * D,), jnp.float32),
            pltpu.VMEM((D, TAIL), jnp.float32),
            pltpu.SemaphoreType.DMA,
            pltpu.SemaphoreType.DMA,
            pltpu.SemaphoreType.DMA,
            pltpu.SemaphoreType.DMA,
            pltpu.SemaphoreType.DMA,
            pltpu.SemaphoreType.DMA,
        ],
    )
    fu, fi, fh = tr_run(W_user.T, W_item.T, W_hist.T)
    Wu = fu.reshape(V, D)
    Wi = fi.reshape(V, D)
    Wh = fh.reshape(V, D)

    run = pl.kernel(
        _sc_body,
        out_type=jax.ShapeDtypeStruct((B, 4 * D), jnp.float32),
        mesh=mesh,
        compiler_params=cpu,
        scratch_types=[
            pltpu.VMEM((C,), jnp.int32),
            pltpu.VMEM((C,), jnp.int32),
            pltpu.VMEM((C,), jnp.float32),
            pltpu.VMEM((RPC,), jnp.int32),
            pltpu.VMEM((C, D), jnp.float32),
            pltpu.VMEM((C, D), jnp.float32),
            pltpu.VMEM((RPC, D), jnp.float32),
            pltpu.VMEM((C,), jnp.float32),
            pltpu.VMEM((C, 4 * D), jnp.float32),
            pltpu.VMEM((D,), jnp.float32),
            pltpu.SemaphoreType.DMA,
        ],
    )
    return run(uid, iid, pricef, hidx, Wu, Wi, Wh, wp)
